# Initial kernel scaffold; baseline (speedup 1.0000x reference)
import jax
import jax.numpy as jnp
from jax.experimental import pallas as pl


def kernel(inputs, embeddings):
    raise NotImplementedError("write your pallas kernel here")


# speedup vs baseline: 1.0923x; 1.0923x over previous
import functools

import jax
import jax.numpy as jnp
from jax import lax
from jax.experimental import pallas as pl
from jax.experimental.pallas import tpu as pltpu
from jax.experimental.pallas import tpu_sc as plsc

EMBED_D = 32
CHUNK = 1024


def _emb_gather_body(table_hbm, idx_hbm, out_hbm, idx_v, rows_v, sem):
    info = plsc.get_sparse_core_info()
    nc, ns = info.num_cores, info.num_subcores
    nw = nc * ns
    wid = lax.axis_index("s") * nc + lax.axis_index("c")
    n = idx_hbm.shape[0]
    b_per_w = n // nw
    base = wid * b_per_w
    nch = b_per_w // CHUNK

    def body(i, carry):
        off = base + i * CHUNK
        pltpu.sync_copy(idx_hbm.at[pl.ds(off, CHUNK)], idx_v)
        pltpu.async_copy(table_hbm.at[idx_v], rows_v, sem).wait()
        pltpu.sync_copy(rows_v, out_hbm.at[pl.ds(off, CHUNK)])
        return carry

    lax.fori_loop(0, nch, body, 0)


def kernel(inputs, embeddings):
    b, s = inputs.shape
    n = b * s
    flat_idx = jnp.reshape(inputs, (n,)).astype(jnp.int32)
    gather = pl.kernel(
        _emb_gather_body,
        mesh=plsc.VectorSubcoreMesh(core_axis_name="c", subcore_axis_name="s"),
        out_type=jax.ShapeDtypeStruct((n, EMBED_D), jnp.float32),
        scratch_types=[
            pltpu.VMEM((CHUNK,), jnp.int32),
            pltpu.VMEM((CHUNK, EMBED_D), jnp.float32),
            pltpu.SemaphoreType.DMA,
        ],
        compiler_params=pltpu.CompilerParams(use_tc_tiling_on_sc=False),
    )
    out = gather(embeddings, flat_idx)
    return jnp.reshape(out, (b, s, EMBED_D))


# speedup vs baseline: 1.4660x; 1.3422x over previous
import functools

import jax
import jax.numpy as jnp
from jax import lax
from jax.experimental import pallas as pl
from jax.experimental.pallas import tpu as pltpu
from jax.experimental.pallas import tpu_sc as plsc

EMBED_D = 32
LANES = 16
BT_PER_SB = 4
SB_IDX = BT_PER_SB * 128


def _emb_body(table_hbm, idxt_hbm, out_hbm, idx_v, rows_v, tiles_v, sem):
    info = plsc.get_sparse_core_info()
    nc = info.num_cores
    nw = nc * info.num_subcores
    wid = lax.axis_index("s") * nc + lax.axis_index("c")

    n_seq = idxt_hbm.shape[0]
    n_batch = idxt_hbm.shape[1]
    n_sb = n_batch // SB_IDX
    total_sb = n_seq * n_sb
    per_w = total_sb // nw

    row_iota = lax.iota(jnp.int32, LANES)

    def body(k, carry):
        sb = wid * per_w + k
        s = sb // n_sb
        bt0 = sb % n_sb
        col0 = bt0 * SB_IDX
        pltpu.sync_copy(idxt_hbm.at[s, pl.ds(col0, SB_IDX)], idx_v)
        cps = []
        for j in range(BT_PER_SB):
            cps.append(
                pltpu.async_copy(
                    table_hbm.at[idx_v.at[pl.ds(j * 128, 128)]],
                    rows_v.at[pl.ds(j * 128, 128)],
                    sem,
                )
            )
        for cp in cps:
            cp.wait()
        for j in range(BT_PER_SB):
            for tr in range(EMBED_D // 8):
                for r in range(8):
                    d = tr * 8 + r
                    cols = jnp.full((LANES,), d, jnp.int32)
                    for lg in range(128 // LANES):
                        row_ids = row_iota + (j * 128 + lg * LANES)
                        tiles_v[tr, j, r, pl.ds(lg * LANES, LANES)] = (
                            plsc.load_gather(rows_v, [row_ids, cols])
                        )
        for tr in range(EMBED_D // 8):
            pltpu.sync_copy(
                tiles_v.at[tr], out_hbm.at[s, tr, pl.ds(bt0 * BT_PER_SB, BT_PER_SB)]
            )
        return carry

    lax.fori_loop(0, per_w, body, 0)


def kernel(inputs, embeddings):
    b, s = inputs.shape
    idxt = jnp.transpose(inputs).astype(jnp.int32)
    emb = pl.kernel(
        _emb_body,
        mesh=plsc.VectorSubcoreMesh(core_axis_name="c", subcore_axis_name="s"),
        out_type=jax.ShapeDtypeStruct((s, EMBED_D // 8, b // 128, 8, 128), jnp.float32),
        scratch_types=[
            pltpu.VMEM((SB_IDX,), jnp.int32),
            pltpu.VMEM((SB_IDX, EMBED_D), jnp.float32),
            pltpu.VMEM((EMBED_D // 8, BT_PER_SB, 8, 128), jnp.float32),
            pltpu.SemaphoreType.DMA,
        ],
        compiler_params=pltpu.CompilerParams(
            use_tc_tiling_on_sc=False, needs_layout_passes=False
        ),
    )
    out5 = emb(embeddings, idxt)
    out = jnp.transpose(out5, (2, 4, 0, 1, 3)).reshape(b, s, EMBED_D)
    return out


# speedup vs baseline: 1.7700x; 1.2073x over previous
import functools

import jax
import jax.numpy as jnp
from jax import lax
from jax.experimental import pallas as pl
from jax.experimental.pallas import tpu as pltpu
from jax.experimental.pallas import tpu_sc as plsc

EMBED_D = 32
LANES = 16
BT_PER_SB = 4
SB_IDX = BT_PER_SB * 128
NBUF = 2


def _emb_body(table_hbm, idxt_hbm, out_hbm, idx_v, rows_v, tiles_v, sems):
    info = plsc.get_sparse_core_info()
    nc = info.num_cores
    nw = nc * info.num_subcores
    wid = lax.axis_index("s") * nc + lax.axis_index("c")

    n_seq = idxt_hbm.shape[0]
    n_batch = idxt_hbm.shape[1]
    n_sb = n_batch // SB_IDX
    total_sb = n_seq * n_sb
    per_w = total_sb // nw

    row_iota = lax.iota(jnp.int32, LANES)

    def fetch(sb, p):
        s = sb // n_sb
        col0 = (sb % n_sb) * SB_IDX
        pltpu.sync_copy(idxt_hbm.at[s, pl.ds(col0, SB_IDX)], idx_v.at[p])
        for j in range(BT_PER_SB):
            pltpu.async_copy(
                table_hbm.at[idx_v.at[p, pl.ds(j * 128, 128)]],
                rows_v.at[p, pl.ds(j * 128, 128)],
                sems.at[p],
            )

    def drain(p):
        for j in range(BT_PER_SB):
            pltpu.make_async_copy(
                table_hbm.at[idx_v.at[p, pl.ds(j * 128, 128)]],
                rows_v.at[p, pl.ds(j * 128, 128)],
                sems.at[p],
            ).wait()

    def process(sb, p):
        @plsc.parallel_loop(0, BT_PER_SB * EMBED_D * (128 // LANES), unroll=8)
        def _(t):
            lg = lax.rem(t, 8)
            d = lax.rem(lax.div(t, 8), EMBED_D)
            j = lax.div(t, 8 * EMBED_D)
            row_ids = row_iota + (j * 128 + lg * LANES)
            cols = jnp.broadcast_to(d, (LANES,))
            v = plsc.load_gather(rows_v.at[p], [row_ids, cols])
            tr = lax.div(d, 8)
            r = lax.rem(d, 8)
            tiles_v[tr, j, r, pl.ds(lg * LANES, LANES)] = v

        s = sb // n_sb
        bt0 = (sb % n_sb) * BT_PER_SB
        for tr in range(EMBED_D // 8):
            pltpu.sync_copy(
                tiles_v.at[tr],
                out_hbm.at[s, tr, pl.ds(bt0, BT_PER_SB)],
            )

    first_sb = wid * per_w
    fetch(first_sb, 0)

    def body(k2, carry):
        for p in range(NBUF):
            sb = first_sb + k2 * NBUF + p
            nxt = sb + 1
            drain(p)

            @pl.when(nxt < first_sb + per_w)
            def _():
                fetch(nxt, (p + 1) % NBUF)

            process(sb, p)
        return carry

    lax.fori_loop(0, per_w // NBUF, body, 0)


def kernel(inputs, embeddings):
    b, s = inputs.shape
    idxt = jnp.transpose(inputs).astype(jnp.int32)
    emb = pl.kernel(
        _emb_body,
        mesh=plsc.VectorSubcoreMesh(core_axis_name="c", subcore_axis_name="s"),
        out_type=jax.ShapeDtypeStruct((s, EMBED_D // 8, b // 128, 8, 128), jnp.float32),
        scratch_types=[
            pltpu.VMEM((NBUF, SB_IDX), jnp.int32),
            pltpu.VMEM((NBUF, SB_IDX, EMBED_D), jnp.float32),
            pltpu.VMEM((EMBED_D // 8, BT_PER_SB, 8, 128), jnp.float32),
            pltpu.SemaphoreType.DMA((NBUF,)),
        ],
        compiler_params=pltpu.CompilerParams(
            use_tc_tiling_on_sc=False, needs_layout_passes=False
        ),
    )
    out5 = emb(embeddings, idxt)
    out = jnp.transpose(out5, (2, 4, 0, 1, 3)).reshape(b, s, EMBED_D)
    return out


# speedup vs baseline: 2.0297x; 1.1468x over previous
import functools

import jax
import jax.numpy as jnp
from jax import lax
from jax.experimental import pallas as pl
from jax.experimental.pallas import tpu as pltpu
from jax.experimental.pallas import tpu_sc as plsc

EMBED_D = 32
LANES = 16
BT_PER_SB = 4
SB_IDX = BT_PER_SB * 128
NBUF = 2


def _emb_body(table_hbm, idxt_hbm, out_hbm, idx_v, rows_v, tiles_v, gsem, osem):
    info = plsc.get_sparse_core_info()
    nc = info.num_cores
    nw = nc * info.num_subcores
    wid = lax.axis_index("s") * nc + lax.axis_index("c")

    n_seq = idxt_hbm.shape[0]
    n_batch = idxt_hbm.shape[1]
    n_sb = n_batch // SB_IDX
    total_sb = n_seq * n_sb
    per_w = total_sb // nw

    row_iota = lax.iota(jnp.int32, LANES)

    def fetch(sb, p):
        s = sb // n_sb
        col0 = (sb % n_sb) * SB_IDX
        pltpu.sync_copy(idxt_hbm.at[s, pl.ds(col0, SB_IDX)], idx_v.at[p])
        pltpu.async_copy(table_hbm.at[idx_v.at[p]], rows_v.at[p], gsem.at[p])

    def drain_gather(p):
        pltpu.make_async_copy(
            table_hbm.at[idx_v.at[p]], rows_v.at[p], gsem.at[p]
        ).wait()

    def out_slices(sb, p):
        s = sb // n_sb
        bt0 = (sb % n_sb) * BT_PER_SB
        return [
            (tiles_v.at[p, tr], out_hbm.at[s, tr, pl.ds(bt0, BT_PER_SB)])
            for tr in range(EMBED_D // 8)
        ]

    def process(sb, p):
        @plsc.parallel_loop(0, BT_PER_SB * (128 // LANES), unroll=2)
        def _(t):
            j = lax.div(t, 8)
            lg = lax.rem(t, 8)
            row_ids = row_iota + (j * 128 + lg * LANES)
            for d in range(EMBED_D):
                cols = jnp.full((LANES,), d, jnp.int32)
                v = plsc.load_gather(rows_v.at[p], [row_ids, cols])
                tiles_v[p, d // 8, j, d % 8, pl.ds(lg * LANES, LANES)] = v

        for src, dst in out_slices(sb, p):
            pltpu.async_copy(src, dst, osem.at[p])

    def drain_out(sb, p):
        for src, dst in out_slices(sb, p):
            pltpu.make_async_copy(src, dst, osem.at[p]).wait()

    first_sb = wid * per_w
    fetch(first_sb, 0)

    def body(k2, carry):
        for p in range(NBUF):
            sb = first_sb + k2 * NBUF + p
            drain_gather(p)

            @pl.when(sb + 1 < first_sb + per_w)
            def _():
                fetch(sb + 1, (p + 1) % NBUF)

            @pl.when(sb - NBUF >= first_sb)
            def _():
                drain_out(sb - NBUF, p)

            process(sb, p)
        return carry

    lax.fori_loop(0, per_w // NBUF, body, 0)
    for p in range(NBUF):
        drain_out(first_sb + per_w - NBUF + p, p)


def kernel(inputs, embeddings):
    b, s = inputs.shape
    idxt = jnp.transpose(inputs).astype(jnp.int32)
    emb = pl.kernel(
        _emb_body,
        mesh=plsc.VectorSubcoreMesh(core_axis_name="c", subcore_axis_name="s"),
        out_type=jax.ShapeDtypeStruct((s, EMBED_D // 8, b // 128, 8, 128), jnp.float32),
        scratch_types=[
            pltpu.VMEM((NBUF, SB_IDX), jnp.int32),
            pltpu.VMEM((NBUF, SB_IDX, EMBED_D), jnp.float32),
            pltpu.VMEM((NBUF, EMBED_D // 8, BT_PER_SB, 8, 128), jnp.float32),
            pltpu.SemaphoreType.DMA((NBUF,)),
            pltpu.SemaphoreType.DMA((NBUF,)),
        ],
        compiler_params=pltpu.CompilerParams(
            use_tc_tiling_on_sc=False, needs_layout_passes=False
        ),
    )
    out5 = emb(embeddings, idxt)
    out = jnp.transpose(out5, (2, 4, 0, 1, 3)).reshape(b, s, EMBED_D)
    return out
